# baseline (speedup 1.0000x reference)
import functools

import jax
import jax.numpy as jnp
from jax import lax
from jax.experimental import pallas as pl
from jax.experimental.pallas import tpu as pltpu
from jax.experimental.pallas import tpu_sc as plsc

NC = 2
NS = 16
NW = NC * NS
CHUNK = 112
NBUF = 4
NSPLIT = 2


@functools.partial(jax.jit, static_argnums=(2, 3, 4))
def _embed(idx, weight, n_chunks, s, d):
    assert n_chunks % NBUF == 0
    n_rounds = n_chunks // NBUF
    spad = CHUNK // 2
    mesh = plsc.VectorSubcoreMesh(core_axis_name="c", subcore_axis_name="s")

    @functools.partial(
        pl.kernel,
        mesh=mesh,
        out_type=jax.ShapeDtypeStruct((NW * n_chunks * 2, s, d), jnp.float32),
        scratch_types=(
            [pltpu.VMEM((n_chunks, CHUNK), jnp.int32)]
            + [pltpu.VMEM((CHUNK, d), jnp.float32) for _ in range(NBUF)]
            + [pltpu.SemaphoreType.DMA for _ in range(2 * NBUF)]
        ),
        compiler_params=pltpu.CompilerParams(use_tc_tiling_on_sc=True),
    )
    def emb(idx_hbm, table_hbm, out_hbm, idx_v, *bufs_and_sems):
        bufs = bufs_and_sems[:NBUF]
        gsem = bufs_and_sems[NBUF:2 * NBUF]
        wsem = bufs_and_sems[2 * NBUF:]
        wid = lax.axis_index("s") * NC + lax.axis_index("c")
        pltpu.sync_copy(idx_hbm.at[wid], idx_v)

        def gather(c, b):
            pltpu.async_copy(table_hbm.at[idx_v.at[c]], bufs[b], gsem[b])

        for b in range(NBUF):
            gather(b, b)

        def round_body(r, carry):
            for b in range(NBUF):
                c = r * NBUF + b
                row = (wid * n_chunks + c) * 2
                pltpu.make_async_copy(
                    table_hbm.at[idx_v.at[c]], bufs[b], gsem[b]).wait()
                pltpu.async_copy(
                    bufs[b].at[pl.ds(0, s)], out_hbm.at[row], wsem[b])
                pltpu.async_copy(
                    bufs[b].at[pl.ds(spad, s)], out_hbm.at[row + 1], wsem[b])
                pltpu.make_async_copy(
                    bufs[b].at[pl.ds(0, s)], out_hbm.at[row], wsem[b]).wait()
                pltpu.make_async_copy(
                    bufs[b].at[pl.ds(spad, s)], out_hbm.at[row + 1],
                    wsem[b]).wait()

                @pl.when(r + 1 < n_rounds)
                def _():
                    gather(c + NBUF, b)
            return carry

        lax.fori_loop(0, n_rounds, round_body, 0)

    return emb(idx, weight)


def kernel(x, weight):
    b0, s = x.shape
    v, d = weight.shape
    spad = CHUNK // 2
    rows_per_w = b0 // (NW * NSPLIT)
    assert b0 % (NW * NSPLIT) == 0 and (rows_per_w * spad) % CHUNK == 0
    n_chunks = rows_per_w * spad // CHUNK
    pad_vals = (jnp.arange(b0 * (spad - s), dtype=jnp.int32) % v
                ).reshape(NSPLIT, NW, rows_per_w, spad - s)
    idx = jnp.concatenate(
        [x.astype(jnp.int32).reshape(NSPLIT, NW, rows_per_w, s), pad_vals],
        axis=-1)
    idx = idx.reshape(NSPLIT, NW, n_chunks, CHUNK)
    bh = b0 // NSPLIT
    out = jnp.zeros((b0, s, d), jnp.float32)
    for h in range(NSPLIT):
        piece = _embed(idx[h], weight, n_chunks, s, d)
        out = lax.dynamic_update_slice(out, piece, (h * bh, 0, 0))
    return out


# speedup vs baseline: 1.7071x; 1.7071x over previous
import functools

import jax
import jax.numpy as jnp
from jax import lax
from jax.experimental import pallas as pl
from jax.experimental.pallas import tpu as pltpu
from jax.experimental.pallas import tpu_sc as plsc

NC = 2
NS = 16
NW = NC * NS
NBUF = 4


@functools.partial(jax.jit, static_argnums=(2, 3))
def _embed(idx, weight, s, d):
    n_chunks = idx.shape[1]
    chunk = idx.shape[2]
    assert n_chunks % NBUF == 0 and chunk == 2 * s
    n_rounds = n_chunks // NBUF
    mesh = plsc.VectorSubcoreMesh(core_axis_name="c", subcore_axis_name="s")

    @functools.partial(
        pl.kernel,
        mesh=mesh,
        out_type=jax.ShapeDtypeStruct((NW * n_chunks * 2, s, d), jnp.float32),
        scratch_types=(
            [pltpu.VMEM((n_chunks, chunk), jnp.int32)]
            + [pltpu.VMEM((chunk, d), jnp.float32) for _ in range(NBUF)]
            + [pltpu.SemaphoreType.DMA for _ in range(2 * NBUF)]
        ),
        compiler_params=pltpu.CompilerParams(use_tc_tiling_on_sc=True),
    )
    def emb(idx_hbm, table_hbm, out_hbm, idx_v, *bufs_and_sems):
        bufs = bufs_and_sems[:NBUF]
        gsem = bufs_and_sems[NBUF:2 * NBUF]
        wsem = bufs_and_sems[2 * NBUF:]
        wid = lax.axis_index("s") * NC + lax.axis_index("c")
        pltpu.sync_copy(idx_hbm.at[wid], idx_v)

        def gather(c, b):
            pltpu.async_copy(table_hbm.at[idx_v.at[c]], bufs[b], gsem[b])

        for b in range(NBUF):
            gather(b, b)

        def round_body(r, carry):
            for b in range(NBUF):
                c = r * NBUF + b
                row = (wid * n_chunks + c) * 2
                pltpu.make_async_copy(
                    table_hbm.at[idx_v.at[c]], bufs[b], gsem[b]).wait()
                pltpu.async_copy(
                    bufs[b].at[pl.ds(0, s)], out_hbm.at[row], wsem[b])
                pltpu.async_copy(
                    bufs[b].at[pl.ds(s, s)], out_hbm.at[row + 1], wsem[b])
                pltpu.make_async_copy(
                    bufs[b].at[pl.ds(0, s)], out_hbm.at[row], wsem[b]).wait()
                pltpu.make_async_copy(
                    bufs[b].at[pl.ds(s, s)], out_hbm.at[row + 1],
                    wsem[b]).wait()

                @pl.when(r + 1 < n_rounds)
                def _():
                    gather(c + NBUF, b)
            return carry

        lax.fori_loop(0, n_rounds, round_body, 0)

    return emb(idx, weight)


def kernel(x, weight):
    b0, s = x.shape
    v, d = weight.shape
    rows_per_w = b0 // NW
    assert b0 % NW == 0 and rows_per_w % 2 == 0
    n_chunks = rows_per_w // 2
    idx = x.astype(jnp.int32).reshape(NW, n_chunks, 2 * s)
    out = _embed(idx, weight, s, d)
    return out.reshape(b0, s, d)
